# Initial kernel scaffold; baseline (speedup 1.0000x reference)
import jax
import jax.numpy as jnp
from jax.experimental import pallas as pl


def kernel(ids, segment_ids, params):
    raise NotImplementedError("write your pallas kernel here")


# speedup vs baseline: 1.7685x; 1.7685x over previous
import functools

import jax
import jax.numpy as jnp
from jax import lax
from jax.experimental import pallas as pl
from jax.experimental.pallas import tpu as pltpu
from jax.experimental.pallas import tpu_sc as plsc

_VOCAB = 1000000
_DIM = 32
_BATCH = 4096
_NNZ = 204800

_NW = 32
_SEG_PER_W = _BATCH // _NW
_CHUNK = 1024
_SUB = 128
_ACC_ROWS = _SEG_PER_W + 2


def _sel16(v, k):
    s = v[0]
    for j in range(1, 16):
        s = jnp.where(k == j, v[j], s)
    return s


def _lower_bound(seg_hbm, probe_ref, target):
    def body(_, carry):
        lo, hi = carry
        m = (lo + hi) // 2
        m8 = pl.multiple_of(jnp.minimum(m & ~7, _NNZ - 16), 8)
        pltpu.sync_copy(seg_hbm.at[pl.ds(m8, 16)], probe_ref)
        v = _sel16(probe_ref[pl.ds(0, 16)], m - m8)
        lt = v < target
        lo = jnp.where(lt, m + 1, lo)
        hi = jnp.where(lt, hi, m)
        return lo, hi

    lo, _ = lax.fori_loop(0, 18, body, (jnp.int32(0), jnp.int32(_NNZ)))
    return lo


def _body(ids_hbm, seg_hbm, params_hbm, out_hbm,
          probe_ref, idx_ref, segv_ref, rows_ref, acc_ref, cnt_ref, sem):
    wid = lax.axis_index("c") * 16 + lax.axis_index("s")
    seg_base = wid * _SEG_PER_W

    def zero_acc(k, _):
        acc_ref[pl.ds(k * 16, 16)] = jnp.zeros((16,), jnp.float32)
        return 0

    lax.fori_loop(0, (_ACC_ROWS * _DIM) // 16, zero_acc, 0)

    def zero_cnt(k, _):
        cnt_ref[k] = 0.0
        return 0

    lax.fori_loop(0, _ACC_ROWS, zero_cnt, 0)

    start = _lower_bound(seg_hbm, probe_ref, seg_base)
    end = _lower_bound(seg_hbm, probe_ref, seg_base + _SEG_PER_W)

    a0 = start & ~7
    e8 = (end + 7) & ~7
    nchunks = (e8 - a0 + _CHUNK - 1) // _CHUNK

    def chunk_body(t, _):
        logical = a0 + t * _CHUNK
        p = pl.multiple_of(
            jnp.minimum(logical, _NNZ - _CHUNK), 8)
        d = logical - p
        m = jnp.minimum(_CHUNK, e8 - logical)

        pltpu.sync_copy(ids_hbm.at[pl.ds(p, _CHUNK)], idx_ref)
        pltpu.sync_copy(seg_hbm.at[pl.ds(p, _CHUNK)], segv_ref)

        copies = []
        for j in range(_CHUNK // _SUB):
            copies.append(pltpu.make_async_copy(
                params_hbm.at[idx_ref.at[pl.ds(j * _SUB, _SUB)]],
                rows_ref.at[pl.ds(j * _SUB, _SUB), :],
                sem,
            ))
        for c in copies:
            c.start()
        for c in copies:
            c.wait()

        lane = lax.broadcasted_iota(jnp.int32, (16,), 0)

        def accum(g, _):
            base = pl.multiple_of(g * 16, 16)
            sv = segv_ref[pl.ds(base, 16)]
            pos = base + lane
            ok = (pos >= d) & (pos < d + m)
            rv = jnp.clip(jnp.where(ok, sv - seg_base, -1), -1, _SEG_PER_W) + 1
            offv = rv * _DIM
            for j in range(16):
                off = offv[j]
                acc_ref[pl.ds(off, 16)] = (
                    acc_ref[pl.ds(off, 16)] + rows_ref[base + j, pl.ds(0, 16)])
                acc_ref[pl.ds(off + 16, 16)] = (
                    acc_ref[pl.ds(off + 16, 16)]
                    + rows_ref[base + j, pl.ds(16, 16)])
                r = rv[j]
                cnt_ref[r] = cnt_ref[r] + 1.0
            return 0

        lax.fori_loop(d // 16, (d + m + 15) // 16, accum, 0)
        return 0

    lax.fori_loop(0, nchunks, chunk_body, 0)

    def finalize(r, _):
        c = cnt_ref[r + 1]
        denom = jnp.maximum(jnp.full((16,), c, jnp.float32), 1.0)
        off = (r + 1) * _DIM
        rows_ref[r, pl.ds(0, 16)] = acc_ref[pl.ds(off, 16)] / denom
        rows_ref[r, pl.ds(16, 16)] = acc_ref[pl.ds(off + 16, 16)] / denom
        return 0

    lax.fori_loop(0, _SEG_PER_W, finalize, 0)

    pltpu.sync_copy(rows_ref.at[pl.ds(0, _SEG_PER_W), :],
                    out_hbm.at[pl.ds(seg_base, _SEG_PER_W), :])


@jax.jit
def _run(ids, segment_ids, params):
    k = functools.partial(
        pl.kernel,
        out_type=jax.ShapeDtypeStruct((_BATCH, _DIM), jnp.float32),
        mesh=plsc.VectorSubcoreMesh(core_axis_name="c", subcore_axis_name="s"),
        compiler_params=pltpu.CompilerParams(use_tc_tiling_on_sc=False),
        scratch_types=[
            pltpu.VMEM((16,), jnp.int32),
            pltpu.VMEM((_CHUNK,), jnp.int32),
            pltpu.VMEM((_CHUNK,), jnp.int32),
            pltpu.VMEM((_CHUNK, _DIM), jnp.float32),
            pltpu.VMEM((_ACC_ROWS * _DIM,), jnp.float32),
            pltpu.SMEM((_ACC_ROWS,), jnp.float32),
            pltpu.SemaphoreType.DMA,
        ],
    )(_body)
    return k(ids, segment_ids, params)


def kernel(ids, segment_ids, params):
    return _run(ids, segment_ids, params)
